# baseline (speedup 1.0000x reference)
import functools

import jax
import jax.numpy as jnp
from jax import lax
from jax.experimental import pallas as pl
from jax.experimental.pallas import tpu as pltpu
from jax.experimental.pallas import tpu_sc as plsc

N = 50000
E = 1600000
D_EDGE = 4
HID = 16
C_OUT = 24
W = 16
TROWS = 32
CNT_COL = 8

NC = 2
NS = 16
NW = NC * NS
EPW = E // NW
CHUNK = 400
NCHUNK = EPW // CHUNK
GROUPS = CHUNK // 16

NPAD = 50176
RPT = NPAD // NS
ZROWS = 196
ZCOPIES = RPT // ZROWS

_mesh = plsc.VectorSubcoreMesh(core_axis_name="c", subcore_axis_name="s")
_sc_params = pltpu.CompilerParams(
    needs_layout_passes=False, use_tc_tiling_on_sc=False
)


@functools.partial(
    pl.kernel,
    out_type=(
        jax.ShapeDtypeStruct((NC, NPAD, W), jnp.float32),
        jax.ShapeDtypeStruct((NC, NPAD, W), jnp.float32),
    ),
    mesh=_mesh,
    compiler_params=_sc_params,
    scratch_types=[
        pltpu.VMEM((N,), jnp.float32),
        pltpu.VMEM((W, CHUNK), jnp.float32),
        pltpu.VMEM((CHUNK, W), jnp.float32),
        pltpu.VMEM((CHUNK,), jnp.int32),
        pltpu.VMEM((CHUNK,), jnp.int32),
        pltpu.VMEM((ZROWS, W), jnp.float32),
        pltpu.VMEM_SHARED((NPAD, W), jnp.float32),
    ],
)
def _sc_scatter(x_hbm, mt_hbm, src_hbm, dst_hbm,
                outlo_hbm, outhi_hbm,
                x_v, mt_v, msg_v, src_v, dst_v, z_v, acc_sh):
    cid = lax.axis_index("c")
    sid = lax.axis_index("s")
    wid = sid * NC + cid
    base = wid * EPW

    pltpu.sync_copy(x_hbm, x_v)

    zeros16 = jnp.zeros((16,), jnp.float32)

    def zrow(r, c):
        z_v[r, pl.ds(0, 16)] = zeros16
        return c

    lax.fori_loop(0, ZROWS, zrow, 0)

    for phase, out_hbm in enumerate((outlo_hbm, outhi_hbm)):
        def zcopy(j, c):
            pltpu.sync_copy(
                z_v, acc_sh.at[pl.ds(sid * RPT + j * ZROWS, ZROWS)]
            )
            return c

        lax.fori_loop(0, ZCOPIES, zcopy, 0)
        plsc.subcore_barrier()

        rows16 = lax.iota(jnp.int32, 16)

        def chunk_body(ci, carry):
            off = base + ci * CHUNK
            pltpu.sync_copy(src_hbm.at[pl.ds(off, CHUNK)], src_v)
            pltpu.sync_copy(dst_hbm.at[pl.ds(off, CHUNK)], dst_v)
            pltpu.sync_copy(
                mt_hbm.at[pl.ds(phase * W, W), pl.ds(off, CHUNK)], mt_v
            )

            def grp(gi, c):
                idx = src_v[pl.ds(gi * 16, 16)]
                xs = plsc.load_gather(x_v, [idx])
                for b in range(16):
                    e = gi * 16 + b
                    sp = lax.gather(
                        xs,
                        jnp.full((16, 1), b, jnp.int32),
                        lax.GatherDimensionNumbers(
                            offset_dims=(),
                            collapsed_slice_dims=(0,),
                            start_index_map=(0,),
                        ),
                        (1,),
                        mode=lax.GatherScatterMode.PROMISE_IN_BOUNDS,
                    )
                    if phase == 1:
                        sp = jnp.where(rows16 == CNT_COL, 1.0, sp)
                    vals = plsc.load_gather(
                        mt_v, [rows16, jnp.full((16,), e, jnp.int32)]
                    )
                    msg_v[e, pl.ds(0, 16)] = vals * sp
                return c

            lax.fori_loop(0, GROUPS, grp, 0)
            pltpu.sync_copy(msg_v, acc_sh.at[dst_v], add=True)
            return carry

        lax.fori_loop(0, NCHUNK, chunk_body, 0)
        plsc.subcore_barrier()
        pltpu.sync_copy(
            acc_sh.at[pl.ds(sid * RPT, RPT)],
            out_hbm.at[cid, pl.ds(sid * RPT, RPT)],
        )


BE = 12800


def _tc_msg_body(eat_ref, w0t_ref, b0_ref, w1t_ref, b1_ref, out_ref):
    h = jnp.maximum(
        jnp.dot(w0t_ref[...], eat_ref[...],
                preferred_element_type=jnp.float32)
        + b0_ref[...],
        0.0,
    )
    out_ref[...] = (
        jnp.dot(w1t_ref[...], h, preferred_element_type=jnp.float32)
        + b1_ref[...]
    )


_tc_msg = pl.pallas_call(
    _tc_msg_body,
    grid=(E // BE,),
    in_specs=[
        pl.BlockSpec((D_EDGE, BE), lambda i: (0, i)),
        pl.BlockSpec((HID, D_EDGE), lambda i: (0, 0)),
        pl.BlockSpec((HID, 1), lambda i: (0, 0)),
        pl.BlockSpec((TROWS, HID), lambda i: (0, 0)),
        pl.BlockSpec((TROWS, 1), lambda i: (0, 0)),
    ],
    out_specs=pl.BlockSpec((TROWS, BE), lambda i: (0, i)),
    out_shape=jax.ShapeDtypeStruct((TROWS, E), jnp.float32),
)


BN = 2000


def _tc_final_body(plo_ref, phi_ref, x_ref, root_ref, bias_ref, out_ref):
    lo = plo_ref[0] + plo_ref[1]
    hi = phi_ref[0] + phi_ref[1]
    s = jnp.concatenate([lo, hi[:, : C_OUT - W]], axis=1)
    cnt = hi[:, CNT_COL:CNT_COL + 1]
    mean = s / jnp.maximum(cnt, 1.0)
    out_ref[...] = mean + x_ref[...] * root_ref[...] + bias_ref[...]


_tc_final = pl.pallas_call(
    _tc_final_body,
    grid=(N // BN,),
    in_specs=[
        pl.BlockSpec((NC, BN, W), lambda i: (0, i, 0)),
        pl.BlockSpec((NC, BN, W), lambda i: (0, i, 0)),
        pl.BlockSpec((BN, 1), lambda i: (i, 0)),
        pl.BlockSpec((1, C_OUT), lambda i: (0, 0)),
        pl.BlockSpec((1, C_OUT), lambda i: (0, 0)),
    ],
    out_specs=pl.BlockSpec((BN, C_OUT), lambda i: (i, 0)),
    out_shape=jax.ShapeDtypeStruct((N, C_OUT), jnp.float32),
)


def kernel(x, edge_index, edge_attr, w0, b0, w1, b1, root, bias):
    src = edge_index[0]
    dst = edge_index[1]
    eat = edge_attr.T
    w0t = w0.T
    w1t_pad = jnp.concatenate(
        [w1.T, jnp.zeros((TROWS - C_OUT, HID), jnp.float32)], axis=0
    )
    b1col = jnp.concatenate(
        [
            b1.reshape(C_OUT, 1),
            jnp.ones((1, 1), jnp.float32),
            jnp.zeros((TROWS - C_OUT - 1, 1), jnp.float32),
        ],
        axis=0,
    )
    theta_t = _tc_msg(eat, w0t, b0.reshape(HID, 1), w1t_pad, b1col)
    p_lo, p_hi = _sc_scatter(x.reshape(N), theta_t, src, dst)
    out = _tc_final(
        p_lo,
        p_hi,
        x,
        root,
        bias.reshape(1, C_OUT),
    )
    return out


# speedup vs baseline: 3.1999x; 3.1999x over previous
import functools

import jax
import jax.numpy as jnp
from jax import lax
from jax.experimental import pallas as pl
from jax.experimental.pallas import tpu as pltpu
from jax.experimental.pallas import tpu_sc as plsc

N = 50000
E = 1600000
D_EDGE = 4
HID = 16
C_OUT = 24
W = 16
CNT_COL = 8

NC = 2
NS = 16
NW = NC * NS
EPW = E // NW
CHUNK = 400
NCHUNK = EPW // CHUNK
GROUPS = CHUNK // 16

NPAD = 50176
RPT = NPAD // NS
ZROWS = 196
ZCOPIES = RPT // ZROWS

ER = E // 8
KR = 1600

_mesh = plsc.VectorSubcoreMesh(core_axis_name="c", subcore_axis_name="s")
_sc_params = pltpu.CompilerParams(
    needs_layout_passes=False, use_tc_tiling_on_sc=False
)


@functools.partial(
    pl.kernel,
    out_type=(
        jax.ShapeDtypeStruct((NC, NPAD, W), jnp.float32),
        jax.ShapeDtypeStruct((NC, NPAD, W), jnp.float32),
    ),
    mesh=_mesh,
    compiler_params=_sc_params,
    scratch_types=[
        pltpu.VMEM((N,), jnp.float32),
        pltpu.VMEM((CHUNK, W), jnp.float32),
        pltpu.VMEM((CHUNK,), jnp.int32),
        pltpu.VMEM((CHUNK,), jnp.int32),
        pltpu.VMEM((ZROWS, W), jnp.float32),
        pltpu.VMEM_SHARED((NPAD, W), jnp.float32),
    ],
)
def _sc_scatter(x_hbm, lo_hbm, hi_hbm, src_hbm, dst_hbm,
                outlo_hbm, outhi_hbm,
                x_v, msg_v, src_v, dst_v, z_v, acc_sh):
    cid = lax.axis_index("c")
    sid = lax.axis_index("s")
    wid = sid * NC + cid
    base = wid * EPW

    pltpu.sync_copy(x_hbm, x_v)

    zeros16 = jnp.zeros((16,), jnp.float32)

    def zrow(r, c):
        z_v[r, pl.ds(0, 16)] = zeros16
        return c

    lax.fori_loop(0, ZROWS, zrow, 0)

    for phase, (msg_hbm, out_hbm) in enumerate(
        ((lo_hbm, outlo_hbm), (hi_hbm, outhi_hbm))
    ):
        def zcopy(j, c):
            pltpu.sync_copy(
                z_v, acc_sh.at[pl.ds(sid * RPT + j * ZROWS, ZROWS)]
            )
            return c

        lax.fori_loop(0, ZCOPIES, zcopy, 0)
        plsc.subcore_barrier()

        rows16 = lax.iota(jnp.int32, 16)

        def chunk_body(ci, carry):
            off = base + ci * CHUNK
            pltpu.sync_copy(src_hbm.at[pl.ds(off, CHUNK)], src_v)
            pltpu.sync_copy(dst_hbm.at[pl.ds(off, CHUNK)], dst_v)
            iblk = off // BE
            rem = off % BE
            kgrp = rem // KR
            row0 = iblk * KR + rem % KR
            pltpu.sync_copy(
                msg_hbm.at[pl.ds(row0, CHUNK), pl.ds(kgrp * W, W)],
                msg_v,
            )

            def grp(gi, c):
                idx = src_v[pl.ds(gi * 16, 16)]
                xs = plsc.load_gather(x_v, [idx])
                for b in range(16):
                    sp = lax.gather(
                        xs,
                        jnp.full((16, 1), b, jnp.int32),
                        lax.GatherDimensionNumbers(
                            offset_dims=(),
                            collapsed_slice_dims=(0,),
                            start_index_map=(0,),
                        ),
                        (1,),
                        mode=lax.GatherScatterMode.PROMISE_IN_BOUNDS,
                    )
                    if phase == 1:
                        sp = jnp.where(rows16 == CNT_COL, 1.0, sp)
                    row = gi * 16 + b
                    msg_v[row, pl.ds(0, 16)] = (
                        msg_v[row, pl.ds(0, 16)] * sp
                    )
                return c

            lax.fori_loop(0, GROUPS, grp, 0)
            pltpu.sync_copy(msg_v, acc_sh.at[dst_v], add=True)
            return carry

        lax.fori_loop(0, NCHUNK, chunk_body, 0)
        plsc.subcore_barrier()
        pltpu.sync_copy(
            acc_sh.at[pl.ds(sid * RPT, RPT)],
            out_hbm.at[cid, pl.ds(sid * RPT, RPT)],
        )


BE = 12800


def _tc_msg_body(ea_ref, w0_ref, b0_ref, w1p_ref, b1p_ref, lo_ref, hi_ref):
    h = jnp.maximum(
        jnp.dot(ea_ref[...], w0_ref[...], preferred_element_type=jnp.float32)
        + b0_ref[...],
        0.0,
    )
    theta = (
        jnp.dot(h, w1p_ref[...], preferred_element_type=jnp.float32)
        + b1p_ref[...]
    )
    lo_ref[...] = jnp.concatenate(
        [theta[KR * k:KR * (k + 1), :W] for k in range(8)], axis=1
    )
    hi_ref[...] = jnp.concatenate(
        [theta[KR * k:KR * (k + 1), W:] for k in range(8)], axis=1
    )


_tc_msg = pl.pallas_call(
    _tc_msg_body,
    grid=(E // BE,),
    in_specs=[
        pl.BlockSpec((BE, D_EDGE), lambda i: (i, 0)),
        pl.BlockSpec((D_EDGE, HID), lambda i: (0, 0)),
        pl.BlockSpec((1, HID), lambda i: (0, 0)),
        pl.BlockSpec((HID, 2 * W), lambda i: (0, 0)),
        pl.BlockSpec((1, 2 * W), lambda i: (0, 0)),
    ],
    out_specs=(
        pl.BlockSpec((BE // 8, 128), lambda i: (i, 0)),
        pl.BlockSpec((BE // 8, 128), lambda i: (i, 0)),
    ),
    out_shape=(
        jax.ShapeDtypeStruct((ER, 128), jnp.float32),
        jax.ShapeDtypeStruct((ER, 128), jnp.float32),
    ),
)


BN = 2000


def _tc_final_body(plo_ref, phi_ref, x_ref, root_ref, bias_ref, out_ref):
    lo = plo_ref[0] + plo_ref[1]
    hi = phi_ref[0] + phi_ref[1]
    s = jnp.concatenate([lo, hi[:, : C_OUT - W]], axis=1)
    cnt = hi[:, CNT_COL:CNT_COL + 1]
    mean = s / jnp.maximum(cnt, 1.0)
    out_ref[...] = mean + x_ref[...] * root_ref[...] + bias_ref[...]


_tc_final = pl.pallas_call(
    _tc_final_body,
    grid=(N // BN,),
    in_specs=[
        pl.BlockSpec((NC, BN, W), lambda i: (0, i, 0)),
        pl.BlockSpec((NC, BN, W), lambda i: (0, i, 0)),
        pl.BlockSpec((BN, 1), lambda i: (i, 0)),
        pl.BlockSpec((1, C_OUT), lambda i: (0, 0)),
        pl.BlockSpec((1, C_OUT), lambda i: (0, 0)),
    ],
    out_specs=pl.BlockSpec((BN, C_OUT), lambda i: (i, 0)),
    out_shape=jax.ShapeDtypeStruct((N, C_OUT), jnp.float32),
)


def kernel(x, edge_index, edge_attr, w0, b0, w1, b1, root, bias):
    src = edge_index[0]
    dst = edge_index[1]
    w1p = jnp.concatenate(
        [w1, jnp.zeros((HID, 2 * W - C_OUT), jnp.float32)], axis=1
    )
    b1p = jnp.concatenate(
        [
            b1,
            jnp.ones((1,), jnp.float32),
            jnp.zeros((2 * W - C_OUT - 1,), jnp.float32),
        ]
    ).reshape(1, 2 * W)
    theta_lo, theta_hi = _tc_msg(
        edge_attr, w0, b0.reshape(1, HID), w1p, b1p
    )
    p_lo, p_hi = _sc_scatter(x.reshape(N), theta_lo, theta_hi, src, dst)
    out = _tc_final(
        p_lo,
        p_hi,
        x,
        root,
        bias.reshape(1, C_OUT),
    )
    return out


# speedup vs baseline: 3.4691x; 1.0841x over previous
import functools

import jax
import jax.numpy as jnp
from jax import lax
from jax.experimental import pallas as pl
from jax.experimental.pallas import tpu as pltpu
from jax.experimental.pallas import tpu_sc as plsc

N = 50000
E = 1600000
D_EDGE = 4
HID = 16
C_OUT = 24
W = 16
CNT_COL = 8

NC = 2
NS = 16
NW = NC * NS
EPW = E // NW
CHUNK = 800
NCHUNK = EPW // CHUNK
GROUPS = CHUNK // 16

NPAD = 50176
RPT = NPAD // NS
ZROWS = 196
ZCOPIES = RPT // ZROWS

ER = E // 8
KR = 1600

_mesh = plsc.VectorSubcoreMesh(core_axis_name="c", subcore_axis_name="s")
_sc_params = pltpu.CompilerParams(
    needs_layout_passes=False, use_tc_tiling_on_sc=False
)


@functools.partial(
    pl.kernel,
    out_type=(
        jax.ShapeDtypeStruct((NC, NPAD, W), jnp.float32),
        jax.ShapeDtypeStruct((NC, NPAD, W), jnp.float32),
    ),
    mesh=_mesh,
    compiler_params=_sc_params,
    scratch_types=[
        pltpu.VMEM((N,), jnp.float32),
        pltpu.VMEM((CHUNK, W), jnp.float32),
        pltpu.VMEM((CHUNK,), jnp.int32),
        pltpu.VMEM((CHUNK,), jnp.int32),
        pltpu.VMEM((ZROWS, W), jnp.float32),
        pltpu.VMEM_SHARED((NPAD, W), jnp.float32),
    ],
)
def _sc_scatter(x_hbm, lo_hbm, hi_hbm, src_hbm, dst_hbm,
                outlo_hbm, outhi_hbm,
                x_v, msg_v, src_v, dst_v, z_v, acc_sh):
    cid = lax.axis_index("c")
    sid = lax.axis_index("s")
    wid = sid * NC + cid
    base = wid * EPW

    pltpu.sync_copy(x_hbm, x_v)

    zeros16 = jnp.zeros((16,), jnp.float32)

    def zrow(r, c):
        z_v[r, pl.ds(0, 16)] = zeros16
        return c

    lax.fori_loop(0, ZROWS, zrow, 0)

    for phase, (msg_hbm, out_hbm) in enumerate(
        ((lo_hbm, outlo_hbm), (hi_hbm, outhi_hbm))
    ):
        def zcopy(j, c):
            pltpu.sync_copy(
                z_v, acc_sh.at[pl.ds(sid * RPT + j * ZROWS, ZROWS)]
            )
            return c

        lax.fori_loop(0, ZCOPIES, zcopy, 0)
        plsc.subcore_barrier()

        rows16 = lax.iota(jnp.int32, 16)

        def chunk_body(ci, carry):
            off = base + ci * CHUNK
            pltpu.sync_copy(src_hbm.at[0, pl.ds(off, CHUNK)], src_v)
            pltpu.sync_copy(dst_hbm.at[0, pl.ds(off, CHUNK)], dst_v)
            iblk = off // BE
            rem = off % BE
            kgrp = rem // KR
            row0 = iblk * KR + rem % KR
            pltpu.sync_copy(
                msg_hbm.at[pl.ds(row0, CHUNK), pl.ds(kgrp * W, W)],
                msg_v,
            )

            def grp(gi, c):
                idx = src_v[pl.ds(gi * 16, 16)]
                xs = plsc.load_gather(x_v, [idx])
                for b in range(16):
                    sp = lax.gather(
                        xs,
                        jnp.full((16, 1), b, jnp.int32),
                        lax.GatherDimensionNumbers(
                            offset_dims=(),
                            collapsed_slice_dims=(0,),
                            start_index_map=(0,),
                        ),
                        (1,),
                        mode=lax.GatherScatterMode.PROMISE_IN_BOUNDS,
                    )
                    if phase == 1:
                        sp = jnp.where(rows16 == CNT_COL, 1.0, sp)
                    row = gi * 16 + b
                    msg_v[row, pl.ds(0, 16)] = (
                        msg_v[row, pl.ds(0, 16)] * sp
                    )
                return c

            lax.fori_loop(0, GROUPS, grp, 0)
            pltpu.sync_copy(msg_v, acc_sh.at[dst_v], add=True)
            return carry

        lax.fori_loop(0, NCHUNK, chunk_body, 0)
        plsc.subcore_barrier()
        pltpu.sync_copy(
            acc_sh.at[pl.ds(sid * RPT, RPT)],
            out_hbm.at[cid, pl.ds(sid * RPT, RPT)],
        )


BE = 12800


def _tc_msg_body(ea_ref, ei_ref, w0_ref, b0_ref, w1p_ref, b1p_ref,
                 lo_ref, hi_ref, src_ref, dst_ref):
    h = jnp.maximum(
        jnp.dot(ea_ref[...], w0_ref[...], preferred_element_type=jnp.float32)
        + b0_ref[...],
        0.0,
    )
    theta = (
        jnp.dot(h, w1p_ref[...], preferred_element_type=jnp.float32)
        + b1p_ref[...]
    )
    lo_ref[...] = jnp.concatenate(
        [theta[KR * k:KR * (k + 1), :W] for k in range(8)], axis=1
    )
    hi_ref[...] = jnp.concatenate(
        [theta[KR * k:KR * (k + 1), W:] for k in range(8)], axis=1
    )
    ei = ei_ref[...]
    src_ref[...] = ei[0:1, :]
    dst_ref[...] = ei[1:2, :]


_tc_msg = pl.pallas_call(
    _tc_msg_body,
    grid=(E // BE,),
    in_specs=[
        pl.BlockSpec((BE, D_EDGE), lambda i: (i, 0)),
        pl.BlockSpec((2, BE), lambda i: (0, i)),
        pl.BlockSpec((D_EDGE, HID), lambda i: (0, 0)),
        pl.BlockSpec((1, HID), lambda i: (0, 0)),
        pl.BlockSpec((HID, 2 * W), lambda i: (0, 0)),
        pl.BlockSpec((1, 2 * W), lambda i: (0, 0)),
    ],
    out_specs=(
        pl.BlockSpec((BE // 8, 128), lambda i: (i, 0)),
        pl.BlockSpec((BE // 8, 128), lambda i: (i, 0)),
        pl.BlockSpec((1, BE), lambda i: (0, i)),
        pl.BlockSpec((1, BE), lambda i: (0, i)),
    ),
    out_shape=(
        jax.ShapeDtypeStruct((ER, 128), jnp.float32),
        jax.ShapeDtypeStruct((ER, 128), jnp.float32),
        jax.ShapeDtypeStruct((1, E), jnp.int32),
        jax.ShapeDtypeStruct((1, E), jnp.int32),
    ),
)


BN = 2000


def _tc_final_body(plo_ref, phi_ref, x_ref, root_ref, bias_ref, out_ref):
    lo = plo_ref[0] + plo_ref[1]
    hi = phi_ref[0] + phi_ref[1]
    s = jnp.concatenate([lo, hi[:, : C_OUT - W]], axis=1)
    cnt = hi[:, CNT_COL:CNT_COL + 1]
    mean = s / jnp.maximum(cnt, 1.0)
    out_ref[...] = mean + x_ref[...] * root_ref[...] + bias_ref[...]


_tc_final = pl.pallas_call(
    _tc_final_body,
    grid=(N // BN,),
    in_specs=[
        pl.BlockSpec((NC, BN, W), lambda i: (0, i, 0)),
        pl.BlockSpec((NC, BN, W), lambda i: (0, i, 0)),
        pl.BlockSpec((BN, 1), lambda i: (i, 0)),
        pl.BlockSpec((1, C_OUT), lambda i: (0, 0)),
        pl.BlockSpec((1, C_OUT), lambda i: (0, 0)),
    ],
    out_specs=pl.BlockSpec((BN, C_OUT), lambda i: (i, 0)),
    out_shape=jax.ShapeDtypeStruct((N, C_OUT), jnp.float32),
)


def kernel(x, edge_index, edge_attr, w0, b0, w1, b1, root, bias):
    w1p = jnp.concatenate(
        [w1, jnp.zeros((HID, 2 * W - C_OUT), jnp.float32)], axis=1
    )
    b1p = jnp.concatenate(
        [
            b1,
            jnp.ones((1,), jnp.float32),
            jnp.zeros((2 * W - C_OUT - 1,), jnp.float32),
        ]
    ).reshape(1, 2 * W)
    theta_lo, theta_hi, src, dst = _tc_msg(
        edge_attr, edge_index, w0, b0.reshape(1, HID), w1p, b1p
    )
    p_lo, p_hi = _sc_scatter(x.reshape(N), theta_lo, theta_hi, src, dst)
    out = _tc_final(
        p_lo,
        p_hi,
        x,
        root,
        bias.reshape(1, C_OUT),
    )
    return out


# speedup vs baseline: 3.6194x; 1.0433x over previous
import functools

import jax
import jax.numpy as jnp
from jax import lax
from jax.experimental import pallas as pl
from jax.experimental.pallas import tpu as pltpu
from jax.experimental.pallas import tpu_sc as plsc

N = 50000
E = 1600000
D_EDGE = 4
HID = 16
C_OUT = 24
W = 16
CNT_COL = 8

NC = 2
NS = 16
NW = NC * NS
EPW = E // NW
CHUNK = 800
NCHUNK = EPW // CHUNK
GROUPS = CHUNK // 16

NPAD = 50176
RPT = NPAD // NS
ZROWS = 196
ZCOPIES = RPT // ZROWS

ER = E // 8
KR = 1600

_mesh = plsc.VectorSubcoreMesh(core_axis_name="c", subcore_axis_name="s")
_sc_params = pltpu.CompilerParams(
    needs_layout_passes=False, use_tc_tiling_on_sc=False
)


@functools.partial(
    pl.kernel,
    out_type=(
        jax.ShapeDtypeStruct((NC, NPAD, W), jnp.float32),
        jax.ShapeDtypeStruct((NC, NPAD, W), jnp.float32),
    ),
    mesh=_mesh,
    compiler_params=_sc_params,
    scratch_types=[
        pltpu.VMEM((N,), jnp.float32),
        pltpu.VMEM((CHUNK, W), jnp.float32),
        pltpu.VMEM((CHUNK,), jnp.int32),
        pltpu.VMEM((CHUNK,), jnp.int32),
        pltpu.VMEM((ZROWS, W), jnp.float32),
        pltpu.VMEM_SHARED((NPAD, W), jnp.float32),
    ],
)
def _sc_scatter(x_hbm, lo_hbm, hi_hbm, src_hbm, dst_hbm,
                outlo_hbm, outhi_hbm,
                x_v, msg_v, src_v, dst_v, z_v, acc_sh):
    cid = lax.axis_index("c")
    sid = lax.axis_index("s")
    wid = sid * NC + cid
    base = wid * EPW

    pltpu.sync_copy(x_hbm, x_v)

    zeros16 = jnp.zeros((16,), jnp.float32)

    def zrow(r, c):
        z_v[r, pl.ds(0, 16)] = zeros16
        return c

    lax.fori_loop(0, ZROWS, zrow, 0)

    for phase, (msg_hbm, out_hbm) in enumerate(
        ((lo_hbm, outlo_hbm), (hi_hbm, outhi_hbm))
    ):
        def zcopy(j, c):
            pltpu.sync_copy(
                z_v, acc_sh.at[pl.ds(sid * RPT + j * ZROWS, ZROWS)]
            )
            return c

        lax.fori_loop(0, ZCOPIES, zcopy, 0)
        plsc.subcore_barrier()

        rows16 = lax.iota(jnp.int32, 16)

        def chunk_body(ci, carry):
            off = base + ci * CHUNK
            pltpu.sync_copy(src_hbm.at[pl.ds(off, CHUNK)], src_v)
            pltpu.sync_copy(dst_hbm.at[pl.ds(off, CHUNK)], dst_v)
            iblk = off // BE
            rem = off % BE
            kgrp = rem // KR
            row0 = iblk * KR + rem % KR
            pltpu.sync_copy(
                msg_hbm.at[pl.ds(row0, CHUNK), pl.ds(kgrp * W, W)],
                msg_v,
            )

            def grp(gi, c):
                idx = src_v[pl.ds(gi * 16, 16)]
                xs = plsc.load_gather(x_v, [idx])
                for b in range(16):
                    sp = lax.gather(
                        xs,
                        jnp.full((16, 1), b, jnp.int32),
                        lax.GatherDimensionNumbers(
                            offset_dims=(),
                            collapsed_slice_dims=(0,),
                            start_index_map=(0,),
                        ),
                        (1,),
                        mode=lax.GatherScatterMode.PROMISE_IN_BOUNDS,
                    )
                    if phase == 1:
                        sp = jnp.where(rows16 == CNT_COL, 1.0, sp)
                    row = gi * 16 + b
                    msg_v[row, pl.ds(0, 16)] = (
                        msg_v[row, pl.ds(0, 16)] * sp
                    )
                return c

            lax.fori_loop(0, GROUPS, grp, 0)
            pltpu.sync_copy(msg_v, acc_sh.at[dst_v], add=True)
            return carry

        lax.fori_loop(0, NCHUNK, chunk_body, 0)
        plsc.subcore_barrier()
        pltpu.sync_copy(
            acc_sh.at[pl.ds(sid * RPT, RPT)],
            out_hbm.at[cid, pl.ds(sid * RPT, RPT)],
        )


BE = 12800


def _tc_msg_body(ea_ref, w0_ref, b0_ref, w1p_ref, b1p_ref,
                 lo_ref, hi_ref):
    h = jnp.maximum(
        jnp.dot(ea_ref[...], w0_ref[...], preferred_element_type=jnp.float32)
        + b0_ref[...],
        0.0,
    )
    theta = (
        jnp.dot(h, w1p_ref[...], preferred_element_type=jnp.float32)
        + b1p_ref[...]
    )
    lo_ref[...] = jnp.concatenate(
        [theta[KR * k:KR * (k + 1), :W] for k in range(8)], axis=1
    )
    hi_ref[...] = jnp.concatenate(
        [theta[KR * k:KR * (k + 1), W:] for k in range(8)], axis=1
    )


_tc_msg = pl.pallas_call(
    _tc_msg_body,
    grid=(E // BE,),
    in_specs=[
        pl.BlockSpec((BE, D_EDGE), lambda i: (i, 0)),
        pl.BlockSpec((D_EDGE, HID), lambda i: (0, 0)),
        pl.BlockSpec((1, HID), lambda i: (0, 0)),
        pl.BlockSpec((HID, 2 * W), lambda i: (0, 0)),
        pl.BlockSpec((1, 2 * W), lambda i: (0, 0)),
    ],
    out_specs=(
        pl.BlockSpec((BE // 8, 128), lambda i: (i, 0)),
        pl.BlockSpec((BE // 8, 128), lambda i: (i, 0)),
    ),
    out_shape=(
        jax.ShapeDtypeStruct((ER, 128), jnp.float32),
        jax.ShapeDtypeStruct((ER, 128), jnp.float32),
    ),
)


BN = 2000


def _tc_final_body(plo_ref, phi_ref, x_ref, root_ref, bias_ref, out_ref):
    lo = plo_ref[0] + plo_ref[1]
    hi = phi_ref[0] + phi_ref[1]
    s = jnp.concatenate([lo, hi[:, : C_OUT - W]], axis=1)
    cnt = hi[:, CNT_COL:CNT_COL + 1]
    mean = s / jnp.maximum(cnt, 1.0)
    out_ref[...] = mean + x_ref[...] * root_ref[...] + bias_ref[...]


_tc_final = pl.pallas_call(
    _tc_final_body,
    grid=(N // BN,),
    in_specs=[
        pl.BlockSpec((NC, BN, W), lambda i: (0, i, 0)),
        pl.BlockSpec((NC, BN, W), lambda i: (0, i, 0)),
        pl.BlockSpec((BN, 1), lambda i: (i, 0)),
        pl.BlockSpec((1, C_OUT), lambda i: (0, 0)),
        pl.BlockSpec((1, C_OUT), lambda i: (0, 0)),
    ],
    out_specs=pl.BlockSpec((BN, C_OUT), lambda i: (i, 0)),
    out_shape=jax.ShapeDtypeStruct((N, C_OUT), jnp.float32),
)


def kernel(x, edge_index, edge_attr, w0, b0, w1, b1, root, bias):
    src = edge_index[0]
    dst = edge_index[1]
    w1p = jnp.concatenate(
        [w1, jnp.zeros((HID, 2 * W - C_OUT), jnp.float32)], axis=1
    )
    b1p = jnp.concatenate(
        [
            b1,
            jnp.ones((1,), jnp.float32),
            jnp.zeros((2 * W - C_OUT - 1,), jnp.float32),
        ]
    ).reshape(1, 2 * W)
    theta_lo, theta_hi = _tc_msg(
        edge_attr, w0, b0.reshape(1, HID), w1p, b1p
    )
    p_lo, p_hi = _sc_scatter(x.reshape(N), theta_lo, theta_hi, src, dst)
    out = _tc_final(
        p_lo,
        p_hi,
        x,
        root,
        bias.reshape(1, C_OUT),
    )
    return out
